# baseline (speedup 1.0000x reference)
import jax, jax.numpy as jnp
from jax.experimental import pallas as pl
from jax.experimental.pallas import tpu as pltpu

_K = 64
_ROWS = 64
_NFB = 8

_INV_PIO2 = 0.6366197723675814
_PIO2_HI = 1.57079637050628662109375
_PIO2_LO = -4.37113900018624283e-8


def _sincos(x):
    n = jnp.round(x * jnp.float32(_INV_PIO2))
    i = n.astype(jnp.int32)
    r = x - n * jnp.float32(_PIO2_HI)
    r = r - n * jnp.float32(_PIO2_LO)
    r2 = r * r
    cos_r = jnp.float32(1.0) + r2 * (
        jnp.float32(-0.5) + r2 * (
            jnp.float32(1.0 / 24) + r2 * (
                jnp.float32(-1.0 / 720) + r2 * jnp.float32(1.0 / 40320))))
    sin_r = r * (jnp.float32(1.0) + r2 * (
        jnp.float32(-1.0 / 6) + r2 * (
            jnp.float32(1.0 / 120) + r2 * jnp.float32(-1.0 / 5040))))
    swap = (i & 1) != 0
    base_c = jnp.where(swap, sin_r, cos_r)
    base_s = jnp.where(swap, cos_r, sin_r)
    sgn_c = jnp.where(((i + 1) & 2) != 0, jnp.float32(-1.0), jnp.float32(1.0))
    sgn_s = jnp.where((i & 2) != 0, jnp.float32(-1.0), jnp.float32(1.0))
    return base_s * sgn_s, base_c * sgn_c


def _spectral(mag, phase):
    power = mag * mag
    rows = mag.shape[0]
    pbits = jax.lax.bitcast_convert_type(power, jnp.int32)
    top16 = (pbits >> 16).astype(jnp.int16)
    low16 = ((pbits & 0xFFFF) - 32768).astype(jnp.int16)

    one16 = jnp.int16(1)
    zero16 = jnp.int16(0)

    def cnt16(mask):
        m = jnp.where(mask, one16, zero16)
        w = m.shape[1]
        while w > 128:
            w //= 2
            m = m[:, :w] + m[:, w:]
        return jnp.sum(m.astype(jnp.int32), axis=1, keepdims=True)

    lo = jnp.zeros((rows, 1), jnp.int32)
    hi = jnp.full((rows, 1), 0x7F80, jnp.int32)
    for _ in range(15):
        mid = lo + ((hi - lo) >> 1)
        c = cnt16(top16 <= mid.astype(jnp.int16))
        pred = c >= _K
        hi = jnp.where(pred, mid, hi)
        lo = jnp.where(pred, lo, mid + 1)
    t16 = lo.astype(jnp.int16)
    cbase = cnt16(top16 < t16)

    lowm = jnp.where(top16 == t16, low16, jnp.int16(0x7FFF))
    need = _K - cbase
    lo2 = jnp.full((rows, 1), -32768, jnp.int32)
    hi2 = jnp.full((rows, 1), 32767, jnp.int32)
    for _ in range(16):
        mid = lo2 + ((hi2 - lo2) >> 1)
        c = cnt16(lowm <= mid.astype(jnp.int16))
        pred = c >= need
        hi2 = jnp.where(pred, mid, hi2)
        lo2 = jnp.where(pred, lo2, mid + 1)

    tbits = (lo.astype(jnp.int32) << 16) | (lo2 + 32768)
    tstar = jax.lax.bitcast_convert_type(tbits, jnp.float32)

    kf = jnp.float32(_K)
    below = (power < tstar).astype(jnp.float32)
    cnt_lt = jnp.sum(below, axis=1, keepdims=True)
    sum_lt = jnp.sum(power * below, axis=1, keepdims=True)
    noise = (sum_lt + (kf - cnt_lt) * tstar) * jnp.float32(1.0 / _K)
    sub = jnp.maximum(mag - noise, jnp.float32(0.0))
    s, c = _sincos(phase)
    return sub * c, sub * s


def _main_body(x_hbm, o_hbm, in_buf, out_buf, in_sem, out_sem):
    g = pl.program_id(0)
    ng = pl.num_programs(0)

    def in_copy(gg, slot):
        b = gg // _NFB
        f0 = (gg % _NFB) * _ROWS
        return pltpu.make_async_copy(
            x_hbm.at[b, :, pl.ds(f0, _ROWS), :],
            in_buf.at[slot], in_sem.at[slot])

    def out_copy(gg, slot):
        b = gg // _NFB
        f0 = (gg % _NFB) * _ROWS
        return pltpu.make_async_copy(
            out_buf.at[slot],
            o_hbm.at[b, :, pl.ds(f0, _ROWS), :], out_sem.at[slot])

    @pl.when(g == 0)
    def _():
        in_copy(0, 0).start()

    in_copy(2 * g + 1, 1).start()
    in_copy(2 * g, 0).wait()

    @pl.when(g >= 1)
    def _():
        out_copy(2 * g - 2, 0).wait()

    re, im = _spectral(in_buf[0, 0], in_buf[0, 1])
    out_buf[0, 0] = re
    out_buf[0, 1] = im
    out_copy(2 * g, 0).start()

    @pl.when(g + 1 < ng)
    def _():
        in_copy(2 * g + 2, 0).start()

    in_copy(2 * g + 1, 1).wait()

    @pl.when(g >= 1)
    def _():
        out_copy(2 * g - 1, 1).wait()

    re2, im2 = _spectral(in_buf[1, 0], in_buf[1, 1])
    out_buf[1, 0] = re2
    out_buf[1, 1] = im2
    out_copy(2 * g + 1, 1).start()

    @pl.when(g == ng - 1)
    def _():
        out_copy(2 * g, 0).wait()
        out_copy(2 * g + 1, 1).wait()


def _tail_body(x_ref, o_prev_ref, o_ref):
    del o_prev_ref
    re, im = _spectral(x_ref[0, 0], x_ref[0, 1])
    o_ref[0, 0] = re
    o_ref[0, 1] = im


def kernel(x):
    b, _, f, t = x.shape
    out_main = pl.pallas_call(
        _main_body,
        grid=(b * _NFB // 2,),
        in_specs=[pl.BlockSpec(memory_space=pl.ANY)],
        out_specs=pl.BlockSpec(memory_space=pl.ANY),
        out_shape=jax.ShapeDtypeStruct(x.shape, x.dtype),
        scratch_shapes=[
            pltpu.VMEM((2, 2, _ROWS, t), jnp.float32),
            pltpu.VMEM((2, 2, _ROWS, t), jnp.float32),
            pltpu.SemaphoreType.DMA((2,)),
            pltpu.SemaphoreType.DMA((2,)),
        ],
        compiler_params=pltpu.CompilerParams(
            dimension_semantics=("arbitrary",),
        ),
    )(x)
    tail_spec = pl.BlockSpec((1, 2, 8, t), lambda i: (i, 0, _NFB * _ROWS // 8, 0))
    return pl.pallas_call(
        _tail_body,
        grid=(b,),
        in_specs=[tail_spec, tail_spec],
        out_specs=tail_spec,
        out_shape=jax.ShapeDtypeStruct(x.shape, x.dtype),
        input_output_aliases={1: 0},
    )(x, out_main)
